# baseline (speedup 1.0000x reference)
import jax
import jax.numpy as jnp
from jax import lax
from jax.experimental import pallas as pl
from jax.experimental.pallas import tpu as pltpu
from jax.experimental.pallas import tpu_sc as plsc

N = 10000
D = 128
H1 = 8
C1 = 128
OUT = 128

NPAD = 10240
E_IN = 320000
E_TOT = E_IN + N
B = 48
NB1 = 432
NB2 = 216
E_PAD = 16 * B * NB1
NBLK = NPAD // 256
ACC_ROWS = 10016
APT = ACC_ROWS // 16
MW = 144

_mesh = plsc.VectorSubcoreMesh(core_axis_name="c", subcore_axis_name="s")


def _proj_body(x_ref, w_ref, s_ref, d_ref, h_ref, ad_ref):
    xb = x_ref[...]
    dcols = []
    for t in range(H1):
        hb = jnp.dot(xb, w_ref[t], preferred_element_type=jnp.float32)
        ascol = jnp.sum(hb * s_ref[t], axis=1, keepdims=True)
        h_ref[t] = jnp.concatenate(
            [hb, jnp.broadcast_to(ascol, (256, MW - C1))], axis=1)
        dcols.append(jnp.sum(hb * d_ref[t], axis=1, keepdims=True))
    ad_ref[...] = jnp.concatenate(dcols, axis=1)


def _tc_project(xp, w1r, a1s, a1d):
    return pl.pallas_call(
        _proj_body,
        grid=(NBLK,),
        in_specs=[
            pl.BlockSpec((256, D), lambda nb: (nb, 0)),
            pl.BlockSpec((H1, D, C1), lambda nb: (0, 0, 0)),
            pl.BlockSpec((H1, 1, C1), lambda nb: (0, 0, 0)),
            pl.BlockSpec((H1, 1, C1), lambda nb: (0, 0, 0)),
        ],
        out_specs=[
            pl.BlockSpec((H1, 256, MW), lambda nb: (0, nb, 0)),
            pl.BlockSpec((256, H1), lambda nb: (nb, 0)),
        ],
        out_shape=[
            jax.ShapeDtypeStruct((H1, NPAD, MW), jnp.float32),
            jax.ShapeDtypeStruct((NPAD, H1), jnp.float32),
        ],
    )(xp, w1r, a1s, a1d)


def _mid_body(num_ref, b1_ref, w2_ref, s2_ref, d2_ref, h2_ref, ad_ref):
    acc = jnp.zeros((256, OUT), jnp.float32)
    for t in range(H1):
        nt = num_ref[t]
        o = nt[:, 0:C1] / (nt[:, C1:C1 + 1] + 1e-16) + b1_ref[t]
        hcol = jnp.where(o > 0.0, o, jnp.exp(jnp.minimum(o, 0.0)) - 1.0)
        acc = acc + jnp.dot(hcol, w2_ref[t], preferred_element_type=jnp.float32)
    as2col = jnp.sum(acc * s2_ref[...], axis=1, keepdims=True)
    h2_ref[...] = jnp.concatenate(
        [acc, jnp.broadcast_to(as2col, (256, MW - C1))], axis=1)
    ad_ref[...] = jnp.sum(acc * d2_ref[...], axis=1, keepdims=True)


def _tc_mid(num1, b1r, w2r, a2s, a2d):
    return pl.pallas_call(
        _mid_body,
        grid=(NBLK,),
        in_specs=[
            pl.BlockSpec((H1, 256, MW), lambda nb: (0, nb, 0)),
            pl.BlockSpec((H1, 1, C1), lambda nb: (0, 0, 0)),
            pl.BlockSpec((H1, C1, OUT), lambda nb: (0, 0, 0)),
            pl.BlockSpec((1, OUT), lambda nb: (0, 0)),
            pl.BlockSpec((1, OUT), lambda nb: (0, 0)),
        ],
        out_specs=[
            pl.BlockSpec((256, MW), lambda nb: (nb, 0)),
            pl.BlockSpec((256, 1), lambda nb: (nb, 0)),
        ],
        out_shape=[
            jax.ShapeDtypeStruct((NPAD, MW), jnp.float32),
            jax.ShapeDtypeStruct((NPAD, 1), jnp.float32),
        ],
    )(num1, b1r, w2r, a2s, a2d)


def _out_body(num_ref, b2_ref, o_ref):
    t = num_ref[0] + num_ref[1]
    o_ref[...] = t[:, 0:OUT] / (t[:, OUT:OUT + 1] + 1e-16) + b2_ref[...]


def _tc_out(num2, b2r):
    return pl.pallas_call(
        _out_body,
        grid=(NBLK,),
        in_specs=[
            pl.BlockSpec((2, 256, MW), lambda nb: (0, nb, 0)),
            pl.BlockSpec((1, OUT), lambda nb: (0, 0)),
        ],
        out_specs=pl.BlockSpec((256, OUT), lambda nb: (nb, 0)),
        out_shape=jax.ShapeDtypeStruct((NPAD, OUT), jnp.float32),
    )(num2, b2r)


def _batch_compute(adv, hrows):
    lane = jnp.arange(16, dtype=jnp.int32)
    ones = jnp.ones((16,), jnp.float32)
    for g in range(B // 16):
        jb = g * 16
        advec = adv[pl.ds(jb, 16)]
        for k in range(16):
            row = jb + k
            av = hrows[row, pl.ds(C1, 16)]
            e = av + advec[k] * ones
            e = jnp.where(e > 0.0, e, 0.2 * e)
            wv = jnp.exp(e)
            for jj in range(C1 // 16):
                sl = pl.ds(16 * jj, 16)
                hrows[row, sl] = hrows[row, sl] * wv
            hrows[row, pl.ds(C1, 16)] = jnp.where(lane == 0, wv, 0.0)


def _make_sc_body(l2):
    nb = NB2 if l2 else NB1
    ept = nb * B

    def body(sdp, adf, hf, num_ref, *refs):
        (sdv0, sdv1, sdv2, dstv0, dstv1, dstv2, dsc0, dsc1, dsc2,
         id0, id1, id2, ihh0, ihh1, ihh2,
         adv0, adv1, adv2, h0, h1r, h2r, acc,
         gs0, gs1, gs2, ss0, ss1, ss2, es0, es1, es2) = refs
        c = lax.axis_index("c")
        s = lax.axis_index("s")
        ebase = ((c * 16 + s) if l2 else s) * ept
        sdvs = (sdv0, sdv1, sdv2)
        dstvs = (dstv0, dstv1, dstv2)
        dscs = (dsc0, dsc1, dsc2)
        advs = (adv0, adv1, adv2)
        hs = (h0, h1r, h2r)
        gss = (gs0, gs1, gs2)
        sss = (ss0, ss1, ss2)
        ess = (es0, es1, es2)
        iads = (id0, id1, id2)
        ihs = (ihh0, ihh1, ihh2)
        z16 = jnp.zeros((16,), jnp.float32)
        nv = jnp.full((16,), N, jnp.int32)

        def idx_and_issue(S, t, base_t):
            iad_, ih_ = iads[S], ihs[S]
            for j in range(B // 16):
                sl = pl.ds(16 * j, 16)
                sd = sdvs[S][sl]
                sv = jax.lax.shift_right_logical(sd, 14)
                dv = sd & 16383
                dstvs[S][sl] = dv
                ih_[sl] = sv + base_t
                iad_[sl] = dv + base_t
            pltpu.async_copy(adf.at[iad_], advs[S], gss[S])
            pltpu.async_copy(hf.at[ih_], hs[S], gss[S])

        def wait_gathers(S):
            iad_, ih_ = iads[S], ihs[S]
            pltpu.make_async_copy(adf.at[iad_], advs[S], gss[S]).wait()
            pltpu.make_async_copy(hf.at[ih_], hs[S], gss[S]).wait()

        def issue_srcdst(S, i):
            off = ebase + i * B
            pltpu.async_copy(sdp.at[pl.ds(off, B)], sdvs[S], ess[S])

        def wait_srcdst(S, i):
            off = ebase + i * B
            pltpu.make_async_copy(sdp.at[pl.ds(off, B)], sdvs[S], ess[S]).wait()

        def wait_scatter(S):
            pltpu.make_async_copy(hs[S], acc.at[dscs[S]], sss[S]).wait()

        def head_pass(t):
            base_t = t * NPAD
            for h in hs:
                @pl.loop(0, B)
                def _(i):
                    for j in range(MW // 16):
                        h[i, pl.ds(16 * j, 16)] = z16
            for j in range(B // 16):
                sl = pl.ds(16 * j, 16)
                dsc0[sl] = nv
                dsc1[sl] = nv
                dsc2[sl] = nv

            @pl.loop(0, APT // B)
            def _(j):
                pltpu.sync_copy(hs[0], acc.at[pl.ds(s * APT + j * B, B)])
            pltpu.sync_copy(hs[0].at[pl.ds(0, APT % B)],
                            acc.at[pl.ds(s * APT + (APT // B) * B, APT % B)])

            plsc.subcore_barrier()
            pltpu.async_copy(hs[0], acc.at[dsc0], sss[0], add=True)
            pltpu.async_copy(hs[1], acc.at[dsc1], sss[1], add=True)
            pltpu.async_copy(hs[2], acc.at[dsc2], sss[2], add=True)
            pltpu.sync_copy(sdp.at[pl.ds(ebase, B)], sdvs[0])
            pltpu.sync_copy(sdp.at[pl.ds(ebase + B, B)], sdvs[1])
            wait_scatter(0)
            idx_and_issue(0, t, base_t)
            wait_scatter(1)
            idx_and_issue(1, t, base_t)
            issue_srcdst(2, 2)

            @pl.loop(0, nb // 3)
            def _(kk):
                i0 = kk * 3
                for jj in range(3):
                    g = jj
                    g2 = (jj + 2) % 3
                    i = i0 + jj
                    wait_gathers(g)
                    for j in range(B // 16):
                        sl = pl.ds(16 * j, 16)
                        dscs[g][sl] = dstvs[g][sl]
                    issue_srcdst(g, i + 3)
                    _batch_compute(advs[g], hs[g])
                    pltpu.async_copy(hs[g], acc.at[dscs[g]], sss[g], add=True)
                    wait_srcdst(g2, i + 2)
                    wait_scatter(g2)
                    idx_and_issue(g2, t, base_t)

            wait_gathers(0)
            wait_gathers(1)
            wait_srcdst(2, nb + 2)
            wait_scatter(2)
            plsc.subcore_barrier()
            dst_off = (c * NPAD if l2 else base_t) + s * APT
            pltpu.sync_copy(acc.at[pl.ds(s * APT, APT)],
                            num_ref.at[pl.ds(dst_off, APT)])
            plsc.subcore_barrier()

        if l2:
            head_pass(jnp.int32(0))
        else:
            @pl.loop(0, H1 // 2)
            def _(hi):
                head_pass(c * (H1 // 2) + hi)

    return body


def _sc_scratch(l2):
    return ([pltpu.VMEM((B,), jnp.int32)] * 15
            + [pltpu.VMEM((B,), jnp.float32)] * 3
            + [pltpu.VMEM((B, MW), jnp.float32)] * 3
            + [pltpu.VMEM_SHARED((ACC_ROWS, MW), jnp.float32)]
            + [pltpu.SemaphoreType.DMA] * 9)


def _sc_edge_l1(sdp, adf, hf):
    kern = pl.kernel(
        _make_sc_body(False),
        out_type=jax.ShapeDtypeStruct((H1 * NPAD, MW), jnp.float32),
        mesh=_mesh,
        compiler_params=pltpu.CompilerParams(use_tc_tiling_on_sc=False),
        scratch_types=_sc_scratch(False),
    )
    return kern(sdp, adf, hf)


def _sc_edge_l2(sdp, adf, hf):
    kern = pl.kernel(
        _make_sc_body(True),
        out_type=jax.ShapeDtypeStruct((2 * NPAD, MW), jnp.float32),
        mesh=_mesh,
        compiler_params=pltpu.CompilerParams(use_tc_tiling_on_sc=False),
        scratch_types=_sc_scratch(True),
    )
    return kern(sdp, adf, hf)


def kernel(x, edge_index, W1, a_src1, a_dst1, b1, W2, a_src2, a_dst2, b2):
    src = edge_index[0].astype(jnp.int32)
    dst = edge_index[1].astype(jnp.int32)
    loops = jnp.arange(N, dtype=jnp.int32)
    pad_n = E_PAD + 3 * B - E_TOT
    padv = N + (jnp.arange(pad_n, dtype=jnp.int32) % 16)
    srcp = jnp.concatenate([src, loops, padv])
    dstp = jnp.concatenate([dst, loops, padv])
    sdp = srcp * 16384 + dstp

    xp = jnp.pad(x, ((0, NPAD - N), (0, 0)))
    w1r = jnp.transpose(W1.reshape(D, H1, C1), (1, 0, 2))
    a1s = a_src1.reshape(H1, 1, C1)
    a1d = a_dst1.reshape(H1, 1, C1)

    h1T, ad1 = _tc_project(xp, w1r, a1s, a1d)
    num1 = _sc_edge_l1(sdp, ad1.T.reshape(-1), h1T.reshape(H1 * NPAD, MW))

    w2r = W2.reshape(H1, C1, OUT)
    b1r = b1.reshape(H1, 1, C1)
    h2, ad2 = _tc_mid(num1.reshape(H1, NPAD, MW), b1r, w2r, a_src2, a_dst2)

    num2 = _sc_edge_l2(sdp, ad2.reshape(-1), h2)
    out = _tc_out(num2.reshape(2, NPAD, MW), b2.reshape(1, OUT))
    return out[:N]


# speedup vs baseline: 1.0551x; 1.0551x over previous
import jax
import jax.numpy as jnp
from jax import lax
from jax.experimental import pallas as pl
from jax.experimental.pallas import tpu as pltpu
from jax.experimental.pallas import tpu_sc as plsc

N = 10000
D = 128
H1 = 8
C1 = 128
OUT = 128

NPAD = 10240
E_IN = 320000
E_TOT = E_IN + N
B = 48
NB1 = 432
NB2 = 216
E_PAD = 16 * B * NB1
NBLK = NPAD // 256
ACC_ROWS = 10016
APT = ACC_ROWS // 16
MW = 144

_mesh = plsc.VectorSubcoreMesh(core_axis_name="c", subcore_axis_name="s")


def _proj_body(x_ref, w_ref, s_ref, d_ref, h_ref, as_ref, ad_ref):
    xb = x_ref[...]
    scols, dcols = [], []
    for t in range(H1):
        hb = jnp.dot(xb, w_ref[t], preferred_element_type=jnp.float32)
        h_ref[t] = hb
        scols.append(jnp.sum(hb * s_ref[t], axis=1, keepdims=True))
        dcols.append(jnp.sum(hb * d_ref[t], axis=1, keepdims=True))
    as_ref[...] = jnp.concatenate(scols, axis=1)
    ad_ref[...] = jnp.concatenate(dcols, axis=1)


def _tc_project(xp, w1r, a1s, a1d):
    return pl.pallas_call(
        _proj_body,
        grid=(NBLK,),
        in_specs=[
            pl.BlockSpec((256, D), lambda nb: (nb, 0)),
            pl.BlockSpec((H1, D, C1), lambda nb: (0, 0, 0)),
            pl.BlockSpec((H1, 1, C1), lambda nb: (0, 0, 0)),
            pl.BlockSpec((H1, 1, C1), lambda nb: (0, 0, 0)),
        ],
        out_specs=[
            pl.BlockSpec((H1, 256, C1), lambda nb: (0, nb, 0)),
            pl.BlockSpec((256, H1), lambda nb: (nb, 0)),
            pl.BlockSpec((256, H1), lambda nb: (nb, 0)),
        ],
        out_shape=[
            jax.ShapeDtypeStruct((H1, NPAD, C1), jnp.float32),
            jax.ShapeDtypeStruct((NPAD, H1), jnp.float32),
            jax.ShapeDtypeStruct((NPAD, H1), jnp.float32),
        ],
    )(xp, w1r, a1s, a1d)


def _mid_body(num_ref, b1_ref, w2_ref, s2_ref, d2_ref, h2_ref, as_ref, ad_ref):
    acc = jnp.zeros((256, OUT), jnp.float32)
    for t in range(H1):
        nt = num_ref[t]
        o = nt[:, 0:C1] / (nt[:, C1:C1 + 1] + 1e-16) + b1_ref[t]
        hcol = jnp.where(o > 0.0, o, jnp.exp(jnp.minimum(o, 0.0)) - 1.0)
        acc = acc + jnp.dot(hcol, w2_ref[t], preferred_element_type=jnp.float32)
    h2_ref[...] = acc
    as_ref[...] = jnp.sum(acc * s2_ref[...], axis=1, keepdims=True)
    ad_ref[...] = jnp.sum(acc * d2_ref[...], axis=1, keepdims=True)


def _tc_mid(num1, b1r, w2r, a2s, a2d):
    return pl.pallas_call(
        _mid_body,
        grid=(NBLK,),
        in_specs=[
            pl.BlockSpec((H1, 256, MW), lambda nb: (0, nb, 0)),
            pl.BlockSpec((H1, 1, C1), lambda nb: (0, 0, 0)),
            pl.BlockSpec((H1, C1, OUT), lambda nb: (0, 0, 0)),
            pl.BlockSpec((1, OUT), lambda nb: (0, 0)),
            pl.BlockSpec((1, OUT), lambda nb: (0, 0)),
        ],
        out_specs=[
            pl.BlockSpec((256, OUT), lambda nb: (nb, 0)),
            pl.BlockSpec((256, 1), lambda nb: (nb, 0)),
            pl.BlockSpec((256, 1), lambda nb: (nb, 0)),
        ],
        out_shape=[
            jax.ShapeDtypeStruct((NPAD, OUT), jnp.float32),
            jax.ShapeDtypeStruct((NPAD, 1), jnp.float32),
            jax.ShapeDtypeStruct((NPAD, 1), jnp.float32),
        ],
    )(num1, b1r, w2r, a2s, a2d)


def _out_body(num_ref, b2_ref, o_ref):
    t = num_ref[0] + num_ref[1]
    o_ref[...] = t[:, 0:OUT] / (t[:, OUT:OUT + 1] + 1e-16) + b2_ref[...]


def _tc_out(num2, b2r):
    return pl.pallas_call(
        _out_body,
        grid=(NBLK,),
        in_specs=[
            pl.BlockSpec((2, 256, MW), lambda nb: (0, nb, 0)),
            pl.BlockSpec((1, OUT), lambda nb: (0, 0)),
        ],
        out_specs=pl.BlockSpec((256, OUT), lambda nb: (nb, 0)),
        out_shape=jax.ShapeDtypeStruct((NPAD, OUT), jnp.float32),
    )(num2, b2r)


def _batch_compute(asv, adv, hrows, msg):
    lane = jnp.arange(16, dtype=jnp.int32)
    ones = jnp.ones((16,), jnp.float32)

    for g in range(B // 16):
        jb = g * 16
        e = asv[pl.ds(jb, 16)] + adv[pl.ds(jb, 16)]
        e = jnp.where(e > 0.0, e, 0.2 * e)
        w16 = jnp.exp(e)
        for k in range(16):
            wv = w16[k] * ones
            row = jb + k
            for jj in range(C1 // 16):
                sl = pl.ds(16 * jj, 16)
                msg[row, sl] = hrows[row, sl] * wv
            msg[row, pl.ds(C1, 16)] = jnp.where(lane == 0, wv, 0.0)


def _make_sc_body(l2):
    nb = NB2 if l2 else NB1
    ept = nb * B

    def body(sdp, asf, adf, hf, num_ref, *refs):
        (sdv0, sdv1, sdv2, dstv0, dstv1, dstv2,
         dsc0, dsc1, dsc2,
         id0, id1, id2, ihh0, ihh1, ihh2,
         asv0, asv1, asv2, adv0, adv1, adv2, h0, h1r, h2r, m0, m1, m2, acc,
         gs0, gs1, gs2, ss0, ss1, ss2, es0, es1, es2) = refs
        c = lax.axis_index("c")
        s = lax.axis_index("s")
        ebase = ((c * 16 + s) if l2 else s) * ept
        sdvs = (sdv0, sdv1, sdv2)
        dstvs = (dstv0, dstv1, dstv2)
        dscs = (dsc0, dsc1, dsc2)
        asvs = (asv0, asv1, asv2)
        advs = (adv0, adv1, adv2)
        hs = (h0, h1r, h2r)
        ms = (m0, m1, m2)
        gss = (gs0, gs1, gs2)
        sss = (ss0, ss1, ss2)
        ess = (es0, es1, es2)
        iads = (id0, id1, id2)
        ihs = (ihh0, ihh1, ihh2)
        z16 = jnp.zeros((16,), jnp.float32)
        nv = jnp.full((16,), N, jnp.int32)

        def idx_and_issue(S, t, base_t):
            iad_, ih_ = iads[S], ihs[S]
            for j in range(B // 16):
                sl = pl.ds(16 * j, 16)
                sd = sdvs[S][sl]
                sv = jax.lax.shift_right_logical(sd, 14)
                dv = sd & 16383
                dstvs[S][sl] = dv
                ih_[sl] = sv + base_t
                iad_[sl] = dv + base_t
            pltpu.async_copy(asf.at[ih_], asvs[S], gss[S])
            pltpu.async_copy(adf.at[iad_], advs[S], gss[S])
            pltpu.async_copy(hf.at[ih_], hs[S], gss[S])

        def wait_gathers(S):
            iad_, ih_ = iads[S], ihs[S]
            pltpu.make_async_copy(asf.at[ih_], asvs[S], gss[S]).wait()
            pltpu.make_async_copy(adf.at[iad_], advs[S], gss[S]).wait()
            pltpu.make_async_copy(hf.at[ih_], hs[S], gss[S]).wait()

        def issue_srcdst(S, i):
            off = ebase + i * B
            pltpu.async_copy(sdp.at[pl.ds(off, B)], sdvs[S], ess[S])

        def wait_srcdst(S, i):
            off = ebase + i * B
            pltpu.make_async_copy(sdp.at[pl.ds(off, B)], sdvs[S], ess[S]).wait()

        def head_pass(t):
            base_t = t * NPAD
            for m in ms:
                @pl.loop(0, B)
                def _(i):
                    for j in range(MW // 16):
                        m[i, pl.ds(16 * j, 16)] = z16
            for j in range(B // 16):
                sl = pl.ds(16 * j, 16)
                dsc0[sl] = nv
                dsc1[sl] = nv
                dsc2[sl] = nv

            @pl.loop(0, APT // B)
            def _(j):
                pltpu.sync_copy(ms[0], acc.at[pl.ds(s * APT + j * B, B)])
            pltpu.sync_copy(ms[0].at[pl.ds(0, APT % B)],
                            acc.at[pl.ds(s * APT + (APT // B) * B, APT % B)])

            plsc.subcore_barrier()
            pltpu.async_copy(ms[0], acc.at[dsc0], sss[0], add=True)
            pltpu.async_copy(ms[1], acc.at[dsc1], sss[1], add=True)
            pltpu.async_copy(ms[2], acc.at[dsc2], sss[2], add=True)
            pltpu.sync_copy(sdp.at[pl.ds(ebase, B)], sdvs[0])
            pltpu.sync_copy(sdp.at[pl.ds(ebase + B, B)], sdvs[1])
            idx_and_issue(0, t, base_t)
            idx_and_issue(1, t, base_t)
            issue_srcdst(2, 2)

            @pl.loop(0, nb // 3)
            def _(kk):
                i0 = kk * 3
                for jj in range(3):
                    g = jj
                    g2 = (jj + 2) % 3
                    i = i0 + jj
                    wait_gathers(g)
                    pltpu.make_async_copy(ms[g], acc.at[dscs[g]], sss[g]).wait()
                    for j in range(B // 16):
                        sl = pl.ds(16 * j, 16)
                        dscs[g][sl] = dstvs[g][sl]
                    issue_srcdst(g, i + 3)
                    _batch_compute(asvs[g], advs[g], hs[g], ms[g])
                    pltpu.async_copy(ms[g], acc.at[dscs[g]], sss[g], add=True)
                    wait_srcdst(g2, i + 2)
                    idx_and_issue(g2, t, base_t)

            wait_gathers(0)
            wait_gathers(1)
            wait_srcdst(2, nb + 2)
            pltpu.make_async_copy(ms[0], acc.at[dscs[0]], sss[0]).wait()
            pltpu.make_async_copy(ms[1], acc.at[dscs[1]], sss[1]).wait()
            pltpu.make_async_copy(ms[2], acc.at[dscs[2]], sss[2]).wait()
            plsc.subcore_barrier()
            dst_off = (c * NPAD if l2 else base_t) + s * APT
            pltpu.sync_copy(acc.at[pl.ds(s * APT, APT)],
                            num_ref.at[pl.ds(dst_off, APT)])
            plsc.subcore_barrier()

        if l2:
            head_pass(jnp.int32(0))
        else:
            @pl.loop(0, H1 // 2)
            def _(hi):
                head_pass(c * (H1 // 2) + hi)

    return body


def _sc_scratch(l2):
    return ([pltpu.VMEM((B,), jnp.int32)] * 15
            + [pltpu.VMEM((B,), jnp.float32)] * 6
            + [pltpu.VMEM((B, C1), jnp.float32)] * 3
            + [pltpu.VMEM((B, MW), jnp.float32)] * 3
            + [pltpu.VMEM_SHARED((ACC_ROWS, MW), jnp.float32)]
            + [pltpu.SemaphoreType.DMA] * 9)


def _sc_edge_l1(sdp, asf, adf, hf):
    kern = pl.kernel(
        _make_sc_body(False),
        out_type=jax.ShapeDtypeStruct((H1 * NPAD, MW), jnp.float32),
        mesh=_mesh,
        compiler_params=pltpu.CompilerParams(use_tc_tiling_on_sc=False),
        scratch_types=_sc_scratch(False),
    )
    return kern(sdp, asf, adf, hf)


def _sc_edge_l2(sdp, asf, adf, hf):
    kern = pl.kernel(
        _make_sc_body(True),
        out_type=jax.ShapeDtypeStruct((2 * NPAD, MW), jnp.float32),
        mesh=_mesh,
        compiler_params=pltpu.CompilerParams(use_tc_tiling_on_sc=False),
        scratch_types=_sc_scratch(True),
    )
    return kern(sdp, asf, adf, hf)


def kernel(x, edge_index, W1, a_src1, a_dst1, b1, W2, a_src2, a_dst2, b2):
    src = edge_index[0].astype(jnp.int32)
    dst = edge_index[1].astype(jnp.int32)
    loops = jnp.arange(N, dtype=jnp.int32)
    pad_n = E_PAD + 3 * B - E_TOT
    padv = N + (jnp.arange(pad_n, dtype=jnp.int32) % 16)
    srcp = jnp.concatenate([src, loops, padv])
    dstp = jnp.concatenate([dst, loops, padv])
    sdp = srcp * 16384 + dstp

    xp = jnp.pad(x, ((0, NPAD - N), (0, 0)))
    w1r = jnp.transpose(W1.reshape(D, H1, C1), (1, 0, 2))
    a1s = a_src1.reshape(H1, 1, C1)
    a1d = a_dst1.reshape(H1, 1, C1)

    h1T, as1, ad1 = _tc_project(xp, w1r, a1s, a1d)
    num1 = _sc_edge_l1(sdp, as1.T.reshape(-1), ad1.T.reshape(-1),
                       h1T.reshape(H1 * NPAD, C1))

    w2r = W2.reshape(H1, C1, OUT)
    b1r = b1.reshape(H1, 1, C1)
    h2, as2, ad2 = _tc_mid(num1.reshape(H1, NPAD, MW), b1r, w2r, a_src2, a_dst2)

    num2 = _sc_edge_l2(sdp, as2.reshape(-1), ad2.reshape(-1), h2)
    out = _tc_out(num2.reshape(2, NPAD, MW), b2.reshape(1, OUT))
    return out[:N]
